# baseline (speedup 1.0000x reference)
import functools

import jax
import jax.numpy as jnp
from jax import lax
from jax.experimental import pallas as pl
from jax.experimental.pallas import tpu as pltpu
from jax.experimental.pallas import tpu_sc as plsc

_K = 64
_NS = 4
_NW = 32
_BN = 10000


def _matmul_body(x_ref, w_ref, y_ref):
    y_ref[...] = jnp.dot(x_ref[...], w_ref[0], preferred_element_type=jnp.float32)


def _rel_transform(x, weights):
    n, d = x.shape
    r, _, h = weights.shape
    nb = n // _BN
    return pl.pallas_call(
        _matmul_body,
        grid=(nb, r),
        in_specs=[
            pl.BlockSpec((_BN, d), lambda i, j: (i, 0)),
            pl.BlockSpec((1, d, h), lambda i, j: (j, 0, 0)),
        ],
        out_specs=pl.BlockSpec((_BN, h), lambda i, j: (j * (n // _BN) + i, 0)),
        out_shape=jax.ShapeDtypeStruct((r * n, h), jnp.float32),
    )(x, weights)


def _add_body(p_ref, o_ref):
    o_ref[...] = p_ref[0] + p_ref[1]


def _combine(partials):
    _, n, h = partials.shape
    return pl.pallas_call(
        _add_body,
        grid=(n // _BN,),
        in_specs=[pl.BlockSpec((2, _BN, h), lambda i: (0, i, 0))],
        out_specs=pl.BlockSpec((_BN, h), lambda i: (i, 0)),
        out_shape=jax.ShapeDtypeStruct((n, h), jnp.float32),
    )(partials)


def _sc_gather_segsum(y, cidx, ptr_pad, n_nodes, blocks, e_real):
    _, h = y.shape
    per_w = blocks * _K
    p_len = ptr_pad.shape[0]
    acc_rows = ((n_nodes + 1 + 127) // 128) * 128
    spare = acc_rows - n_nodes
    steps = (n_nodes + 2).bit_length()
    z_per = acc_rows // 16
    o_per = (n_nodes // 16) // 8 * 8
    o_tail = n_nodes - o_per * 16

    mesh = plsc.VectorSubcoreMesh(core_axis_name="c", subcore_axis_name="s")

    @functools.partial(
        pl.kernel,
        mesh=mesh,
        out_type=jax.ShapeDtypeStruct((2, n_nodes, h), jnp.float32),
        scratch_types=[
            [pltpu.VMEM((_K,), jnp.int32)] * _NS,
            [pltpu.VMEM((_K,), jnp.int32)] * _NS,
            pltpu.VMEM((p_len,), jnp.int32),
            [pltpu.VMEM((_K, h), jnp.float32)] * _NS,
            pltpu.VMEM_SHARED((acc_rows, h), jnp.float32),
            [pltpu.SemaphoreType.DMA] * _NS,
            [pltpu.SemaphoreType.DMA] * _NS,
        ],
        compiler_params=pltpu.CompilerParams(needs_layout_passes=False),
    )
    def k(y_hbm, cidx_hbm, ptr_hbm, out_hbm,
          ci, sg, ptr_v, rows, acc_sh, gsem, ssem):
        cid = lax.axis_index("c")
        sid = lax.axis_index("s")
        base_e = (cid * 16 + sid) * per_w
        lanes = jnp.arange(16, dtype=jnp.int32)

        def compute_seg(eb, sg_buf):
            nv = _K // 16
            ev = [eb + (lanes + 16 * v) for v in range(nv)]
            lo = [jnp.zeros((16,), jnp.int32) for _ in range(nv)]
            hi = [jnp.full((16,), n_nodes + 1, jnp.int32) for _ in range(nv)]
            for _t in range(steps):
                for v in range(nv):
                    mid = (lo[v] + hi[v]) >> 1
                    pm = plsc.load_gather(ptr_v, [mid])
                    cond = pm <= ev[v]
                    lo[v] = jnp.where(cond, mid + 1, lo[v])
                    hi[v] = jnp.where(cond, hi[v], mid)
            for v in range(nv):
                s16 = lo[v] - 1
                s16 = jnp.where(
                    ev[v] >= e_real,
                    n_nodes + lax.rem(ev[v] - e_real, jnp.int32(spare)),
                    s16)
                sg_buf[pl.ds(16 * v, 16)] = s16

        def stage_fire(b_idx, s):
            eb = pl.multiple_of(base_e + b_idx * _K, 8)
            pltpu.sync_copy(cidx_hbm.at[pl.ds(eb, _K)], ci[s])
            pltpu.async_copy(y_hbm.at[ci[s]], rows[s], gsem[s])
            compute_seg(eb, sg[s])

        def wait_gather(s):
            pltpu.make_async_copy(y_hbm.at[ci[s]], rows[s], gsem[s]).wait()

        def fire_scatter(s):
            pltpu.async_copy(rows[s], acc_sh.at[sg[s]], ssem[s], add=True)

        def wait_scatter(s):
            pltpu.make_async_copy(rows[s], acc_sh.at[sg[s]], ssem[s]).wait()

        pltpu.sync_copy(ptr_hbm, ptr_v)
        for b in range(_NS - 1):
            stage_fire(b, b)

        def zrow(i, carry):
            for j in range(h // 16):
                rows[_NS - 1][i, pl.ds(j * 16, 16)] = jnp.zeros(
                    (16,), jnp.float32)
            return carry

        lax.fori_loop(0, _K, zrow, 0)
        z0 = pl.multiple_of(sid * z_per, 8)
        for t in range(z_per // _K):
            pltpu.sync_copy(rows[_NS - 1], acc_sh.at[pl.ds(z0 + t * _K, _K)])
        rem = z_per % _K
        if rem:
            base = (z_per // _K) * _K
            pltpu.sync_copy(rows[_NS - 1].at[pl.ds(0, rem)],
                            acc_sh.at[pl.ds(z0 + base, rem)])
        plsc.subcore_barrier()

        wait_gather(0)
        fire_scatter(0)
        stage_fire(_NS - 1, _NS - 1)
        for b in range(1, _NS - 1):
            wait_gather(b)
            fire_scatter(b)

        def spin(j, carry):
            for u in range(_NS):
                s = u
                pm1 = (u + _NS - 1) % _NS
                wait_scatter(s)
                stage_fire(_NS * j + _NS + u, s)
                wait_gather(pm1)
                fire_scatter(pm1)
            return carry

        lax.fori_loop(0, (blocks - _NS) // _NS, spin, 0)
        wait_gather(_NS - 1)
        fire_scatter(_NS - 1)
        for s in range(_NS):
            wait_scatter(s)
        plsc.subcore_barrier()

        r0 = pl.multiple_of(sid * o_per, 8)
        pltpu.sync_copy(acc_sh.at[pl.ds(r0, o_per)],
                        out_hbm.at[cid, pl.ds(r0, o_per)])
        if o_tail:
            t0 = o_per * 16

            @pl.when(sid == 0)
            def _copy_tail():
                pltpu.sync_copy(acc_sh.at[pl.ds(t0, o_tail)],
                                out_hbm.at[cid, pl.ds(t0, o_tail)])

    return k(y, cidx, ptr_pad)


def kernel(x, weights, ptr, idx, edge_types, num_node):
    n, _ = x.shape
    e = idx.shape[0]

    cidx = edge_types.astype(jnp.int32) * n + idx.astype(jnp.int32)

    chunk = _NW * _K
    blocks = (e + chunk - 1) // chunk
    blocks += (-blocks) % _NS
    blocks = max(blocks, _NS)
    e_pad = blocks * chunk
    if e_pad != e:
        pad = e_pad - e
        k = jnp.arange(pad, dtype=jnp.int32)
        cidx = jnp.concatenate([cidx, k % jnp.int32(weights.shape[0] * n)])

    p_len = ((n + 1 + 63) // 64) * 64
    ptr_pad = jnp.concatenate(
        [ptr.astype(jnp.int32),
         jnp.full((p_len - (n + 1),), e, jnp.int32)])

    y = _rel_transform(x, weights)
    partials = _sc_gather_segsum(y, cidx, ptr_pad, n, blocks, e)
    return _combine(partials)


# speedup vs baseline: 1.1775x; 1.1775x over previous
import functools

import jax
import jax.numpy as jnp
from jax import lax
from jax.experimental import pallas as pl
from jax.experimental.pallas import tpu as pltpu
from jax.experimental.pallas import tpu_sc as plsc

_K = 96
_NW = 32
_BN = 10000


def _matmul_body(x_ref, w_ref, y_ref):
    y_ref[...] = jnp.dot(x_ref[...], w_ref[0], preferred_element_type=jnp.float32)


def _rel_transform(x, weights):
    n, d = x.shape
    r, _, h = weights.shape
    nb = n // _BN
    return pl.pallas_call(
        _matmul_body,
        grid=(nb, r),
        in_specs=[
            pl.BlockSpec((_BN, d), lambda i, j: (i, 0)),
            pl.BlockSpec((1, d, h), lambda i, j: (j, 0, 0)),
        ],
        out_specs=pl.BlockSpec((_BN, h), lambda i, j: (j * (n // _BN) + i, 0)),
        out_shape=jax.ShapeDtypeStruct((r * n, h), jnp.float32),
    )(x, weights)


def _add_body(p_ref, o_ref):
    o_ref[...] = p_ref[0] + p_ref[1]


def _combine(partials):
    _, n, h = partials.shape
    return pl.pallas_call(
        _add_body,
        grid=(n // _BN,),
        in_specs=[pl.BlockSpec((2, _BN, h), lambda i: (0, i, 0))],
        out_specs=pl.BlockSpec((_BN, h), lambda i: (i, 0)),
        out_shape=jax.ShapeDtypeStruct((n, h), jnp.float32),
    )(partials)


def _sc_gather_segsum(y, cidx, ptr_pad, n_nodes, blocks, e_real):
    _, h = y.shape
    per_w = blocks * _K
    p_len = ptr_pad.shape[0]
    acc_rows = ((n_nodes + 1 + 127) // 128) * 128
    spare = acc_rows - n_nodes
    steps = (n_nodes + 2).bit_length()
    z_per = acc_rows // 16
    o_per = (n_nodes // 16) // 8 * 8
    o_tail = n_nodes - o_per * 16

    mesh = plsc.VectorSubcoreMesh(core_axis_name="c", subcore_axis_name="s")

    @functools.partial(
        pl.kernel,
        mesh=mesh,
        out_type=jax.ShapeDtypeStruct((2, n_nodes, h), jnp.float32),
        scratch_types=[
            [pltpu.VMEM((_K,), jnp.int32)] * 3,
            [pltpu.VMEM((_K,), jnp.int32)] * 3,
            pltpu.VMEM((p_len,), jnp.int32),
            [pltpu.VMEM((_K, h), jnp.float32)] * 3,
            pltpu.VMEM_SHARED((acc_rows, h), jnp.float32),
            [pltpu.SemaphoreType.DMA] * 3,
            [pltpu.SemaphoreType.DMA] * 3,
        ],
        compiler_params=pltpu.CompilerParams(needs_layout_passes=False),
    )
    def k(y_hbm, cidx_hbm, ptr_hbm, out_hbm,
          ci, sg, ptr_v, rows, acc_sh, gsem, ssem):
        cid = lax.axis_index("c")
        sid = lax.axis_index("s")
        base_e = (cid * 16 + sid) * per_w
        lanes = jnp.arange(16, dtype=jnp.int32)

        def compute_seg(eb, sg_buf):
            nv = _K // 16
            ev = [eb + (lanes + 16 * v) for v in range(nv)]
            lo = [jnp.zeros((16,), jnp.int32) for _ in range(nv)]
            hi = [jnp.full((16,), n_nodes + 1, jnp.int32) for _ in range(nv)]
            for _t in range(steps):
                for v in range(nv):
                    mid = (lo[v] + hi[v]) >> 1
                    pm = plsc.load_gather(ptr_v, [mid])
                    cond = pm <= ev[v]
                    lo[v] = jnp.where(cond, mid + 1, lo[v])
                    hi[v] = jnp.where(cond, hi[v], mid)
            for v in range(nv):
                s16 = lo[v] - 1
                s16 = jnp.where(
                    ev[v] >= e_real,
                    n_nodes + lax.rem(ev[v] - e_real, jnp.int32(spare)),
                    s16)
                sg_buf[pl.ds(16 * v, 16)] = s16

        def stage_fire(b_idx, s):
            eb = pl.multiple_of(base_e + b_idx * _K, 8)
            pltpu.sync_copy(cidx_hbm.at[pl.ds(eb, _K)], ci[s])
            pltpu.async_copy(y_hbm.at[ci[s]], rows[s], gsem[s])
            compute_seg(eb, sg[s])

        def wait_gather(s):
            pltpu.make_async_copy(y_hbm.at[ci[s]], rows[s], gsem[s]).wait()

        def fire_scatter(s):
            pltpu.async_copy(rows[s], acc_sh.at[sg[s]], ssem[s], add=True)

        def wait_scatter(s):
            pltpu.make_async_copy(rows[s], acc_sh.at[sg[s]], ssem[s]).wait()

        pltpu.sync_copy(ptr_hbm, ptr_v)
        stage_fire(0, 0)
        stage_fire(1, 1)

        def zrow(i, carry):
            for j in range(h // 16):
                rows[2][i, pl.ds(j * 16, 16)] = jnp.zeros((16,), jnp.float32)
            return carry

        lax.fori_loop(0, _K, zrow, 0)
        z0 = pl.multiple_of(sid * z_per, 8)
        for t in range(z_per // _K):
            pltpu.sync_copy(rows[2], acc_sh.at[pl.ds(z0 + t * _K, _K)])
        rem = z_per % _K
        if rem:
            base = (z_per // _K) * _K
            pltpu.sync_copy(rows[2].at[pl.ds(0, rem)],
                            acc_sh.at[pl.ds(z0 + base, rem)])
        plsc.subcore_barrier()

        wait_gather(0)
        fire_scatter(0)
        stage_fire(2, 2)
        wait_gather(1)
        fire_scatter(1)

        def triple(j, carry):
            for u in range(3):
                s = u
                pm1 = (u + 2) % 3
                wait_scatter(s)
                stage_fire(3 * j + 3 + u, s)
                wait_gather(pm1)
                fire_scatter(pm1)
            return carry

        lax.fori_loop(0, (blocks - 3) // 3, triple, 0)
        wait_gather(2)
        fire_scatter(2)
        wait_scatter(0)
        wait_scatter(1)
        wait_scatter(2)
        plsc.subcore_barrier()

        r0 = pl.multiple_of(sid * o_per, 8)
        pltpu.sync_copy(acc_sh.at[pl.ds(r0, o_per)],
                        out_hbm.at[cid, pl.ds(r0, o_per)])
        if o_tail:
            t0 = o_per * 16

            @pl.when(sid == 0)
            def _copy_tail():
                pltpu.sync_copy(acc_sh.at[pl.ds(t0, o_tail)],
                                out_hbm.at[cid, pl.ds(t0, o_tail)])

    return k(y, cidx, ptr_pad)


def kernel(x, weights, ptr, idx, edge_types, num_node):
    n, _ = x.shape
    e = idx.shape[0]

    cidx = edge_types.astype(jnp.int32) * n + idx.astype(jnp.int32)

    chunk = _NW * _K
    blocks = (e + chunk - 1) // chunk
    blocks += (-blocks) % 3
    blocks = max(blocks, 3)
    e_pad = blocks * chunk
    if e_pad != e:
        pad = e_pad - e
        k = jnp.arange(pad, dtype=jnp.int32)
        cidx = jnp.concatenate([cidx, k % jnp.int32(weights.shape[0] * n)])

    p_len = ((n + 1 + 63) // 64) * 64
    ptr_pad = jnp.concatenate(
        [ptr.astype(jnp.int32),
         jnp.full((p_len - (n + 1),), e, jnp.int32)])

    y = _rel_transform(x, weights)
    partials = _sc_gather_segsum(y, cidx, ptr_pad, n, blocks, e)
    return _combine(partials)
